# Initial kernel scaffold; baseline (speedup 1.0000x reference)
import jax
import jax.numpy as jnp
from jax.experimental import pallas as pl


def kernel(features, adj, W1, b1, Wmu, bmu, Wls, bls):
    raise NotImplementedError("write your pallas kernel here")


# speedup vs baseline: 1.4189x; 1.4189x over previous
import functools

import jax
import jax.numpy as jnp
from jax.experimental import pallas as pl

_TILE = 200


def _xform_kernel(x_ref, w_ref, out_ref):
    out_ref[0] = jnp.dot(x_ref[...], w_ref[0],
                         preferred_element_type=jnp.float32)


def _xform(x, w):
    m, d_in, d_out = w.shape
    n = x.shape[0]
    return pl.pallas_call(
        _xform_kernel,
        grid=(m,),
        in_specs=[
            pl.BlockSpec((n, d_in), lambda i: (0, 0)),
            pl.BlockSpec((1, d_in, d_out), lambda i: (i, 0, 0)),
        ],
        out_specs=pl.BlockSpec((1, n, d_out), lambda i: (i, 0, 0)),
        out_shape=jax.ShapeDtypeStruct((m, n, d_out), jnp.float32),
    )(x, w)


def _prop_kernel(adj_ref, h_ref, b_ref, out_ref, *, relu):
    acc = jnp.dot(adj_ref[0], h_ref[0], preferred_element_type=jnp.float32)
    acc = acc + jnp.dot(adj_ref[1], h_ref[1],
                        preferred_element_type=jnp.float32)
    acc = acc + b_ref[...]
    if relu:
        acc = jnp.maximum(acc, 0.0)
    out_ref[...] = acc


def _propagate(adj, h, b2d, relu):
    n = adj.shape[1]
    d = h.shape[2]
    return pl.pallas_call(
        functools.partial(_prop_kernel, relu=relu),
        grid=(n // _TILE,),
        in_specs=[
            pl.BlockSpec((2, _TILE, n), lambda i: (0, i, 0)),
            pl.BlockSpec((2, n, d), lambda i: (0, 0, 0)),
            pl.BlockSpec((1, d), lambda i: (0, 0)),
        ],
        out_specs=pl.BlockSpec((_TILE, d), lambda i: (i, 0)),
        out_shape=jax.ShapeDtypeStruct((n, d), jnp.float32),
    )(adj, h, b2d)


def kernel(features, adj, W1, b1, Wmu, bmu, Wls, bls):
    z = bmu.shape[0]
    h1 = _xform(features, W1)
    x = _propagate(adj, h1, b1.reshape(1, -1), relu=True)
    w_heads = jnp.concatenate([Wmu, Wls], axis=-1)
    b_heads = jnp.concatenate([bmu, bls]).reshape(1, -1)
    h2 = _xform(x, w_heads)
    out = _propagate(adj, h2, b_heads, relu=False)
    return out[:, :z], out[:, z:]


# speedup vs baseline: 1.4754x; 1.0398x over previous
import functools

import jax
import jax.numpy as jnp
from jax.experimental import pallas as pl
from jax.experimental.pallas import tpu as pltpu

_TILE = 200


def _pass1_kernel(feat_ref, w_ref, b_ref, adj_ref, out_ref, h_ref):
    @pl.when(pl.program_id(0) == 0)
    def _():
        h_ref[0] = jnp.dot(feat_ref[...], w_ref[0],
                           preferred_element_type=jnp.float32)
        h_ref[1] = jnp.dot(feat_ref[...], w_ref[1],
                           preferred_element_type=jnp.float32)

    acc = jnp.dot(adj_ref[0], h_ref[0], preferred_element_type=jnp.float32)
    acc = acc + jnp.dot(adj_ref[1], h_ref[1],
                        preferred_element_type=jnp.float32)
    out_ref[...] = jnp.maximum(acc + b_ref[...], 0.0)


def _pass2_kernel(x_ref, w_ref, b_ref, adj_ref, zm_ref, zl_ref, h_ref, *, z):
    @pl.when(pl.program_id(0) == 0)
    def _():
        h_ref[0] = jnp.dot(x_ref[...], w_ref[0],
                           preferred_element_type=jnp.float32)
        h_ref[1] = jnp.dot(x_ref[...], w_ref[1],
                           preferred_element_type=jnp.float32)

    acc = jnp.dot(adj_ref[0], h_ref[0], preferred_element_type=jnp.float32)
    acc = acc + jnp.dot(adj_ref[1], h_ref[1],
                        preferred_element_type=jnp.float32)
    acc = acc + b_ref[...]
    zm_ref[...] = acc[:, :z]
    zl_ref[...] = acc[:, z:]


def kernel(features, adj, W1, b1, Wmu, bmu, Wls, bls):
    n = features.shape[0]
    d_in = features.shape[1]
    hid = W1.shape[2]
    z = bmu.shape[0]
    grid = (n // _TILE,)

    x = pl.pallas_call(
        _pass1_kernel,
        grid=grid,
        in_specs=[
            pl.BlockSpec((n, d_in), lambda i: (0, 0)),
            pl.BlockSpec((2, d_in, hid), lambda i: (0, 0, 0)),
            pl.BlockSpec((1, hid), lambda i: (0, 0)),
            pl.BlockSpec((2, _TILE, n), lambda i: (0, i, 0)),
        ],
        out_specs=pl.BlockSpec((_TILE, hid), lambda i: (i, 0)),
        out_shape=jax.ShapeDtypeStruct((n, hid), jnp.float32),
        scratch_shapes=[pltpu.VMEM((2, n, hid), jnp.float32)],
    )(features, W1, b1.reshape(1, -1), adj)

    w_heads = jnp.concatenate([Wmu, Wls], axis=-1)
    b_heads = jnp.concatenate([bmu, bls]).reshape(1, -1)

    z_mean, z_log_std = pl.pallas_call(
        functools.partial(_pass2_kernel, z=z),
        grid=grid,
        in_specs=[
            pl.BlockSpec((n, hid), lambda i: (0, 0)),
            pl.BlockSpec((2, hid, 2 * z), lambda i: (0, 0, 0)),
            pl.BlockSpec((1, 2 * z), lambda i: (0, 0)),
            pl.BlockSpec((2, _TILE, n), lambda i: (0, i, 0)),
        ],
        out_specs=[
            pl.BlockSpec((_TILE, z), lambda i: (i, 0)),
            pl.BlockSpec((_TILE, z), lambda i: (i, 0)),
        ],
        out_shape=[
            jax.ShapeDtypeStruct((n, z), jnp.float32),
            jax.ShapeDtypeStruct((n, z), jnp.float32),
        ],
        scratch_shapes=[pltpu.VMEM((2, n, 2 * z), jnp.float32)],
    )(x, w_heads, b_heads, adj)

    return z_mean, z_log_std


# speedup vs baseline: 1.4816x; 1.0042x over previous
import functools

import jax
import jax.numpy as jnp
from jax.experimental import pallas as pl
from jax.experimental.pallas import tpu as pltpu

_TILE = 200


def _fused_kernel(feat_ref, w1_ref, b1_ref, wh_ref, bh_ref, adj_ref,
                  zm_ref, zl_ref, h_ref, x_ref, *, z, tile):
    p = pl.program_id(0)
    i = pl.program_id(1)

    @pl.when((p == 0) & (i == 0))
    def _():
        h_ref[0] = jnp.dot(feat_ref[...], w1_ref[0],
                           preferred_element_type=jnp.float32)
        h_ref[1] = jnp.dot(feat_ref[...], w1_ref[1],
                           preferred_element_type=jnp.float32)

    @pl.when((p == 1) & (i == 0))
    def _():
        h_ref[0] = jnp.dot(x_ref[...], wh_ref[0],
                           preferred_element_type=jnp.float32)
        h_ref[1] = jnp.dot(x_ref[...], wh_ref[1],
                           preferred_element_type=jnp.float32)

    acc = jnp.dot(adj_ref[0], h_ref[0], preferred_element_type=jnp.float32)
    acc = acc + jnp.dot(adj_ref[1], h_ref[1],
                        preferred_element_type=jnp.float32)

    @pl.when(p == 0)
    def _():
        x_ref[pl.ds(i * tile, tile), :] = jnp.maximum(
            acc + b1_ref[...], 0.0)

    @pl.when(p == 1)
    def _():
        out = acc + bh_ref[...]
        zm_ref[...] = out[:, :z]
        zl_ref[...] = out[:, z:]


def kernel(features, adj, W1, b1, Wmu, bmu, Wls, bls):
    n = features.shape[0]
    d_in = features.shape[1]
    hid = W1.shape[2]
    z = bmu.shape[0]

    w_heads = jnp.concatenate([Wmu, Wls], axis=-1)
    b_heads = jnp.concatenate([bmu, bls]).reshape(1, -1)

    z_mean, z_log_std = pl.pallas_call(
        functools.partial(_fused_kernel, z=z, tile=_TILE),
        grid=(2, n // _TILE),
        in_specs=[
            pl.BlockSpec((n, d_in), lambda p, i: (0, 0)),
            pl.BlockSpec((2, d_in, hid), lambda p, i: (0, 0, 0)),
            pl.BlockSpec((1, hid), lambda p, i: (0, 0)),
            pl.BlockSpec((2, hid, 2 * z), lambda p, i: (0, 0, 0)),
            pl.BlockSpec((1, 2 * z), lambda p, i: (0, 0)),
            pl.BlockSpec((2, _TILE, n), lambda p, i: (0, i, 0)),
        ],
        out_specs=[
            pl.BlockSpec((_TILE, z), lambda p, i: (p * i, 0)),
            pl.BlockSpec((_TILE, z), lambda p, i: (p * i, 0)),
        ],
        out_shape=[
            jax.ShapeDtypeStruct((n, z), jnp.float32),
            jax.ShapeDtypeStruct((n, z), jnp.float32),
        ],
        scratch_shapes=[
            pltpu.VMEM((2, n, hid), jnp.float32),
            pltpu.VMEM((n, hid), jnp.float32),
        ],
    )(features, W1, b1.reshape(1, -1), w_heads, b_heads, adj)

    return z_mean, z_log_std


# speedup vs baseline: 1.5031x; 1.0145x over previous
import functools

import jax
import jax.numpy as jnp
from jax.experimental import pallas as pl
from jax.experimental.pallas import tpu as pltpu

_TILE = 200


def _fused_kernel(feat_ref, w1_ref, b1_ref, wh_ref, bh_ref, adj_ref,
                  zm_ref, zl_ref, h1_ref, h2_ref, *, z, tile):
    p = pl.program_id(0)
    i = pl.program_id(1)

    @pl.when((p == 0) & (i == 0))
    def _():
        h1_ref[0] = jnp.dot(feat_ref[...], w1_ref[0],
                            preferred_element_type=jnp.float32)
        h1_ref[1] = jnp.dot(feat_ref[...], w1_ref[1],
                            preferred_element_type=jnp.float32)

    @pl.when(p == 0)
    def _():
        acc = jnp.dot(adj_ref[0], h1_ref[0],
                      preferred_element_type=jnp.float32)
        acc = acc + jnp.dot(adj_ref[1], h1_ref[1],
                            preferred_element_type=jnp.float32)
        xt = jnp.maximum(acc + b1_ref[...], 0.0)
        h2_ref[0, pl.ds(i * tile, tile), :] = jnp.dot(
            xt, wh_ref[0], preferred_element_type=jnp.float32)
        h2_ref[1, pl.ds(i * tile, tile), :] = jnp.dot(
            xt, wh_ref[1], preferred_element_type=jnp.float32)

    @pl.when(p == 1)
    def _():
        acc = jnp.dot(adj_ref[0], h2_ref[0],
                      preferred_element_type=jnp.float32)
        acc = acc + jnp.dot(adj_ref[1], h2_ref[1],
                            preferred_element_type=jnp.float32)
        out = acc + bh_ref[...]
        zm_ref[...] = out[:, :z]
        zl_ref[...] = out[:, z:]


def kernel(features, adj, W1, b1, Wmu, bmu, Wls, bls):
    n = features.shape[0]
    d_in = features.shape[1]
    hid = W1.shape[2]
    z = bmu.shape[0]

    w_heads = jnp.concatenate([Wmu, Wls], axis=-1)
    b_heads = jnp.concatenate([bmu, bls]).reshape(1, -1)

    z_mean, z_log_std = pl.pallas_call(
        functools.partial(_fused_kernel, z=z, tile=_TILE),
        grid=(2, n // _TILE),
        in_specs=[
            pl.BlockSpec((n, d_in), lambda p, i: (0, 0)),
            pl.BlockSpec((2, d_in, hid), lambda p, i: (0, 0, 0)),
            pl.BlockSpec((1, hid), lambda p, i: (0, 0)),
            pl.BlockSpec((2, hid, 2 * z), lambda p, i: (0, 0, 0)),
            pl.BlockSpec((1, 2 * z), lambda p, i: (0, 0)),
            pl.BlockSpec((2, _TILE, n), lambda p, i: (0, i, 0)),
        ],
        out_specs=[
            pl.BlockSpec((_TILE, z), lambda p, i: (p * i, 0)),
            pl.BlockSpec((_TILE, z), lambda p, i: (p * i, 0)),
        ],
        out_shape=[
            jax.ShapeDtypeStruct((n, z), jnp.float32),
            jax.ShapeDtypeStruct((n, z), jnp.float32),
        ],
        scratch_shapes=[
            pltpu.VMEM((2, n, hid), jnp.float32),
            pltpu.VMEM((2, n, 2 * z), jnp.float32),
        ],
    )(features, W1, b1.reshape(1, -1), w_heads, b_heads, adj)

    return z_mean, z_log_std
